# baseline (speedup 1.0000x reference)
import functools

import jax
import jax.numpy as jnp
from jax import lax
from jax.experimental import pallas as pl
from jax.experimental.pallas import tpu as pltpu
from jax.experimental.pallas import tpu_sc as plsc

B = 2
NPB = 1_000_000
H, W = 37, 120
PIX = H * W
DUMP = B * PIX
TBL = 8896

NC, NS, L = 2, 16, 16
NW = NC * NS

PW = 62_496
GW = PW // L
CHUNK_G = 434
CHUNK_P = CHUNK_G * L
NCHUNK = GW // CHUNK_G
EX_BASE = 16 * PW

_GATHER_DNUMS = lax.GatherDimensionNumbers(
    offset_dims=(), collapsed_slice_dims=(0,), start_index_map=(0,))


def _vgather(v, idx):
    return lax.gather(v, idx, _GATHER_DNUMS, slice_sizes=(1,),
                      mode=lax.GatherScatterMode.PROMISE_IN_BOUNDS)


def _make_sc_kernel():
    mesh = plsc.VectorSubcoreMesh(core_axis_name="c", subcore_axis_name="s")

    @functools.partial(
        pl.kernel,
        mesh=mesh,
        compiler_params=pltpu.CompilerParams(needs_layout_passes=False),
        out_type=[
            jax.ShapeDtypeStruct((NW, TBL), jnp.int32),
            jax.ShapeDtypeStruct((NW, TBL), jnp.float32),
        ],
        scratch_types=[
            pltpu.VMEM((8 * CHUNK_P,), jnp.float32),
            pltpu.VMEM((TBL,), jnp.int32),
            pltpu.VMEM((TBL,), jnp.float32),
            pltpu.VMEM((16,), jnp.float32),
            pltpu.SemaphoreType.DMA,
            pltpu.SemaphoreType.DMA,
        ],
    )
    def sc_kernel(stacked, trans_hbm, n_out, z_out,
                  bufs, n_tbl, z_tbl, tv, sem0, sem1):
        sems = (sem0, sem1)
        wid = lax.axis_index("c") * NS + lax.axis_index("s")
        w_batch = wid // 16
        w_loc = wid % 16
        row0 = w_batch * 4
        p0w = w_loc * PW

        lane = lax.iota(jnp.int32, L)
        nxt_idx = jnp.minimum(lane + 1, L - 1).reshape(L, 1)
        last_lane = lane == (L - 1)
        neg1 = jnp.full((L,), -1, jnp.int32)

        def init_body(i, _):
            n_tbl[pl.ds(i * L, L)] = neg1
            return 0
        lax.fori_loop(0, TBL // L, init_body, 0)

        pltpu.sync_copy(trans_hbm, tv)
        tvec = tv[...]
        def coef(j):
            return _vgather(tvec, jnp.full((L, 1), j, jnp.int32))
        t00, t01, t02, t03 = coef(0), coef(1), coef(2), coef(3)
        t10, t11, t12, t13 = coef(4), coef(5), coef(6), coef(7)
        t20, t21, t22, t23 = coef(8), coef(9), coef(10), coef(11)
        boffv = jnp.broadcast_to(w_batch * PIX, (L,)).astype(jnp.int32)

        def group(r0, r1, r2, r3, goff, gid0):
            p0 = r0[pl.ds(goff, L)]
            p1 = r1[pl.ds(goff, L)]
            p2 = r2[pl.ds(goff, L)]
            p3 = r3[pl.ds(goff, L)]
            x_n = t00 * p0 + t01 * p1 + t02 * p2 + t03 * p3
            y_n = t10 * p0 + t11 * p1 + t12 * p2 + t13 * p3
            z = t20 * p0 + t21 * p1 + t22 * p2 + t23 * p3
            x = jnp.clip(x_n / z, 0.0, float(H - 1))
            y = jnp.clip(y_n / z, 0.0, float(W - 1))
            xi = x.astype(jnp.int32)
            yi = y.astype(jnp.int32)
            gid = gid0 + lane
            pid = jnp.clip(xi * W + yi, 0, PIX - 1) + boffv
            pid = jnp.where(z > 0.0, pid, DUMP)
            key = pid * L + lane
            skey, sgid = plsc.sort_key_val(key, gid)
            _, sz = plsc.sort_key_val(key, z)
            spid = jnp.right_shift(skey, 4)
            is_end = jnp.logical_or(spid != _vgather(spid, nxt_idx),
                                    last_lane)
            plsc.store_scatter(n_tbl, [spid], sgid, mask=is_end)
            plsc.store_scatter(z_tbl, [spid], sz, mask=is_end)

        def bslice(par, j, n=CHUNK_P):
            return bufs.at[pl.ds((par * 4 + j) * CHUNK_P, n)]

        def start(c, par):
            off = p0w + c * CHUNK_P
            for j in range(4):
                pltpu.async_copy(
                    stacked.at[pl.ds((row0 + j) * NPB + off, CHUNK_P)],
                    bslice(par, j), sems[par])

        def wait(c, par):
            off = p0w + c * CHUNK_P
            for j in range(4):
                pltpu.make_async_copy(
                    stacked.at[pl.ds((row0 + j) * NPB + off, CHUNK_P)],
                    bslice(par, j), sems[par]).wait()

        start(0, 0)
        for c in range(NCHUNK):
            par = c % 2
            if c + 1 < NCHUNK:
                start(c + 1, 1 - par)
            wait(c, par)
            r0, r1, r2, r3 = (bslice(par, j) for j in range(4))
            p_base = p0w + c * CHUNK_P
            def group_body(g, _):
                group(r0, r1, r2, r3, g * L, p_base + g * L)
                return 0
            lax.fori_loop(0, CHUNK_G, group_body, 0)

        @pl.when(w_loc < (NPB - EX_BASE) // L)
        def _():
            ex0 = EX_BASE + w_loc * L
            for j in range(4):
                pltpu.sync_copy(
                    stacked.at[pl.ds((row0 + j) * NPB + ex0, L)],
                    bslice(0, j, L))
            r0, r1, r2, r3 = (bslice(0, j, L) for j in range(4))
            group(r0, r1, r2, r3, 0, ex0)

        pltpu.sync_copy(n_tbl, n_out.at[wid])
        pltpu.sync_copy(z_tbl, z_out.at[wid])

    return sc_kernel


def _tc_merge(n_all, z_all):
    def body(n_ref, z_ref, o_ref):
        n = n_ref[...]
        z = z_ref[...]
        bn = jnp.max(n, axis=0, keepdims=True)
        zz = jnp.sum(jnp.where(n == bn, z, 0.0), axis=0, keepdims=True)
        o_ref[...] = jnp.where(bn >= 0, zz, 0.0)

    return pl.pallas_call(
        body,
        out_shape=jax.ShapeDtypeStruct((1, TBL), jnp.float32),
    )(n_all, z_all)


@jax.jit
def kernel(inputs, trans):
    stacked = jnp.transpose(inputs, (0, 2, 1)).reshape(-1)
    t16 = jnp.pad(trans.reshape(-1), (0, 16 - trans.size))
    n_all, z_all = _make_sc_kernel()(stacked, t16)
    merged = _tc_merge(n_all, z_all)
    return merged[0, :B * PIX].reshape(B, H, W)


# speedup vs baseline: 3.7329x; 3.7329x over previous
import functools

import jax
import jax.numpy as jnp
from jax import lax
from jax.experimental import pallas as pl
from jax.experimental.pallas import tpu as pltpu
from jax.experimental.pallas import tpu_sc as plsc

B = 2
NPB = 1_000_000
H, W = 37, 120
PIX = H * W
DUMP = B * PIX
TBL = 8896

NC, NS, L = 2, 16, 16
NW = NC * NS

SC_PTS = 999_936
PW = 62_464
CHUNK_P = 7_808
CHUNK_G = CHUNK_P // L
NCHUNK = PW // CHUNK_P
W15_EXTRA = SC_PTS - 16 * PW
TAIL = NPB - SC_PTS

_GATHER_DNUMS = lax.GatherDimensionNumbers(
    offset_dims=(), collapsed_slice_dims=(0,), start_index_map=(0,))


def _vgather(v, idx):
    return lax.gather(v, idx, _GATHER_DNUMS, slice_sizes=(1,),
                      mode=lax.GatherScatterMode.PROMISE_IN_BOUNDS)


def _bf16_round(v):
    b = plsc.bitcast(v, jnp.int32)
    r = (b + 32767 + (jnp.right_shift(b, 16) & 1)) & jnp.int32(-65536)
    return plsc.bitcast(r, jnp.float32)


def _make_sc_kernel():
    mesh = plsc.VectorSubcoreMesh(core_axis_name="c", subcore_axis_name="s")

    @functools.partial(
        pl.kernel,
        mesh=mesh,
        compiler_params=pltpu.CompilerParams(needs_layout_passes=False),
        out_type=[
            jax.ShapeDtypeStruct((NW, TBL), jnp.int32),
            jax.ShapeDtypeStruct((NW, TBL), jnp.float32),
        ],
        scratch_types=[
            pltpu.VMEM((4, CHUNK_P), jnp.float32),
            pltpu.VMEM((4, CHUNK_P), jnp.float32),
            pltpu.VMEM((4, W15_EXTRA), jnp.float32),
            pltpu.VMEM((TBL,), jnp.int32),
            pltpu.VMEM((TBL,), jnp.float32),
            pltpu.VMEM((16,), jnp.float32),
            pltpu.SemaphoreType.DMA,
            pltpu.SemaphoreType.DMA,
        ],
    )
    def sc_kernel(xt, trans_hbm, n_out, z_out,
                  buf0, buf1, exbuf, n_tbl, z_tbl, tv, sem0, sem1):
        bufs = (buf0, buf1)
        sems = (sem0, sem1)
        wid = lax.axis_index("c") * NS + lax.axis_index("s")
        w_batch = wid // 16
        w_loc = wid % 16
        p0w = w_loc * PW

        lane = lax.iota(jnp.int32, L)
        nxt_idx = jnp.minimum(lane + 1, L - 1).reshape(L, 1)
        last_lane = lane == (L - 1)
        neg1 = jnp.full((L,), -1, jnp.int32)

        def init_body(i, _):
            n_tbl[pl.ds(i * L, L)] = neg1
            return 0
        lax.fori_loop(0, TBL // L, init_body, 0)

        pltpu.sync_copy(trans_hbm, tv)
        tvec = tv[...]
        def coef(j):
            return _vgather(tvec, jnp.full((L, 1), j, jnp.int32))
        t00, t01, t02, t03 = (_bf16_round(coef(j)) for j in range(4))
        t10, t11, t12, t13 = (_bf16_round(coef(j)) for j in range(4, 8))
        t20, t21, t22, t23 = (_bf16_round(coef(j)) for j in range(8, 12))
        boffv = jnp.broadcast_to(w_batch * PIX, (L,)).astype(jnp.int32)

        def group(buf, goff, gid0):
            p0 = _bf16_round(buf[0, pl.ds(goff, L)])
            p1 = _bf16_round(buf[1, pl.ds(goff, L)])
            p2 = _bf16_round(buf[2, pl.ds(goff, L)])
            p3 = _bf16_round(buf[3, pl.ds(goff, L)])
            x_n = t00 * p0 + t01 * p1 + t02 * p2 + t03 * p3
            y_n = t10 * p0 + t11 * p1 + t12 * p2 + t13 * p3
            z = t20 * p0 + t21 * p1 + t22 * p2 + t23 * p3
            x = jnp.clip(x_n / z, 0.0, float(H - 1))
            y = jnp.clip(y_n / z, 0.0, float(W - 1))
            xi = x.astype(jnp.int32)
            yi = y.astype(jnp.int32)
            gid = gid0 + lane
            pid = jnp.clip(xi * W + yi, 0, PIX - 1) + boffv
            pid = jnp.where(z > 0.0, pid, DUMP)
            key = pid * L + lane
            skey, sgid = plsc.sort_key_val(key, gid)
            _, sz = plsc.sort_key_val(key, z)
            spid = jnp.right_shift(skey, 4)
            is_end = jnp.logical_or(spid != _vgather(spid, nxt_idx),
                                    last_lane)
            plsc.store_scatter(n_tbl, [spid], sgid, mask=is_end)
            plsc.store_scatter(z_tbl, [spid], sz, mask=is_end)

        def start(off, par, dst=None, n=CHUNK_P):
            pltpu.async_copy(xt.at[w_batch, :, pl.ds(off, n)],
                             dst if dst is not None else bufs[par],
                             sems[par])

        def wait(off, par, dst=None, n=CHUNK_P):
            pltpu.make_async_copy(xt.at[w_batch, :, pl.ds(off, n)],
                                  dst if dst is not None else bufs[par],
                                  sems[par]).wait()

        def run_chunk(buf, p_base, n_groups, unroll=4):
            def group_body(g, _):
                for u in range(unroll):
                    gu = g * unroll + u
                    group(buf, gu * L, p_base + gu * L)
                return 0
            assert n_groups % unroll == 0
            lax.fori_loop(0, n_groups // unroll, group_body, 0)

        start(p0w, 0)
        for c in range(NCHUNK):
            par = c % 2
            if c + 1 < NCHUNK:
                start(p0w + (c + 1) * CHUNK_P, 1 - par)
            wait(p0w + c * CHUNK_P, par)
            run_chunk(bufs[par], p0w + c * CHUNK_P, CHUNK_G)

        @pl.when(w_loc == 15)
        def _():
            ex0 = 16 * PW
            start(ex0, 1, dst=exbuf, n=W15_EXTRA)
            wait(ex0, 1, dst=exbuf, n=W15_EXTRA)
            run_chunk(exbuf, ex0, W15_EXTRA // L)

        pltpu.sync_copy(n_tbl, n_out.at[wid])
        pltpu.sync_copy(z_tbl, z_out.at[wid])

    return sc_kernel


def _tc_merge(n_all, z_all, tail, trans):
    def body(n_ref, z_ref, tail_ref, t_ref, o_ref):
        n = n_ref[...]
        z = z_ref[...]
        bn = jnp.max(n, axis=0, keepdims=True)
        zz = jnp.sum(jnp.where(n == bn, z, 0.0), axis=0, keepdims=True)
        depth = jnp.where(bn >= 0, zz, 0.0)
        pix_iota = lax.broadcasted_iota(jnp.int32, (1, TBL), 1)
        def bfr(s):
            return s.astype(jnp.bfloat16).astype(jnp.float32)
        tb = [[bfr(t_ref[i, j]) for j in range(4)] for i in range(3)]
        for b in range(B):
            for k in range(TAIL):
                c0 = bfr(tail_ref[b, k, 0])
                c1 = bfr(tail_ref[b, k, 1])
                c2 = bfr(tail_ref[b, k, 2])
                c3 = bfr(tail_ref[b, k, 3])
                x_n = (tb[0][0] * c0 + tb[0][1] * c1
                       + tb[0][2] * c2 + tb[0][3] * c3)
                y_n = (tb[1][0] * c0 + tb[1][1] * c1
                       + tb[1][2] * c2 + tb[1][3] * c3)
                zv = (tb[2][0] * c0 + tb[2][1] * c1
                      + tb[2][2] * c2 + tb[2][3] * c3)
                x = jnp.clip(x_n / zv, 0.0, float(H - 1))
                y = jnp.clip(y_n / zv, 0.0, float(W - 1))
                pid = jnp.clip(x.astype(jnp.int32) * W
                               + y.astype(jnp.int32), 0, PIX - 1) + b * PIX
                hit = jnp.logical_and(pix_iota == pid, zv > 0.0)
                depth = jnp.where(hit, zv, depth)
        o_ref[...] = depth

    return pl.pallas_call(
        body,
        in_specs=[
            pl.BlockSpec(memory_space=pltpu.VMEM),
            pl.BlockSpec(memory_space=pltpu.VMEM),
            pl.BlockSpec(memory_space=pltpu.SMEM),
            pl.BlockSpec(memory_space=pltpu.SMEM),
        ],
        out_shape=jax.ShapeDtypeStruct((1, TBL), jnp.float32),
    )(n_all, z_all, tail, trans)


@jax.jit
def kernel(inputs, trans):
    xt = jnp.transpose(inputs, (0, 2, 1))
    tail = inputs[:, SC_PTS:, :]
    t16 = jnp.pad(trans.reshape(-1), (0, 16 - trans.size))
    n_all, z_all = _make_sc_kernel()(xt, t16)
    merged = _tc_merge(n_all, z_all, tail, trans)
    return merged[0, :B * PIX].reshape(B, H, W)
